# baseline (speedup 1.0000x reference)
import functools

import jax
import jax.numpy as jnp
from jax import lax
from jax.experimental import pallas as pl
from jax.experimental.pallas import tpu as pltpu
from jax.experimental.pallas import tpu_sc as plsc

N = 10000
K = 16
V = 128
F = 16
D = V * F
NW = 32
NP = 10240
NODES_W = NP // NW
C = 8
CK = C * K
NCHUNK = NODES_W // C
BM = 512


def _mask_body(vint_ref, vnh_ref, isint_ref, vi_ref, vn_ref):
    m = (isint_ref[...] == 1).astype(jnp.float32)
    vi_ref[...] = vint_ref[...] * m
    vn_ref[...] = vnh_ref[...] * (1.0 - m)


def _masked_vertices(vint, vnh, isint):
    return pl.pallas_call(
        _mask_body,
        grid=(NP // BM,),
        in_specs=[
            pl.BlockSpec((BM, V), lambda i: (i, 0)),
            pl.BlockSpec((BM, V), lambda i: (i, 0)),
            pl.BlockSpec((BM, 1), lambda i: (i, 0)),
        ],
        out_specs=[
            pl.BlockSpec((BM, V), lambda i: (i, 0)),
            pl.BlockSpec((BM, V), lambda i: (i, 0)),
        ],
        out_shape=[jax.ShapeDtypeStruct((NP, V), jnp.float32)] * 2,
    )(vint, vnh, isint)


def _bag_body(vi_hbm, vn_hbm, nhi_hbm, ini_hbm, nhe_hbm, ine_hbm,
              bag_i_hbm, bag_n_hbm, idx_all, e_all, rows_a, rows_b,
              out_a, out_b, table_sh, gsem_a, gsem_b, osem_a, osem_b):
    cid = lax.axis_index("c")
    sid = lax.axis_index("s")
    wid = sid * 2 + cid
    base = wid * NODES_W
    rpt = NP // 16
    for idxf, ef, tab, outb in ((nhi_hbm, nhe_hbm, vi_hbm, bag_i_hbm),
                                (ini_hbm, ine_hbm, vn_hbm, bag_n_hbm)):
        plsc.subcore_barrier()
        pltpu.sync_copy(tab.at[pl.ds(sid * rpt, rpt)],
                        table_sh.at[pl.ds(sid * rpt, rpt)])
        plsc.subcore_barrier()

        def gather(t, rows, sem):
            pltpu.async_copy(table_sh.at[idx_all.at[pl.ds(t * CK, CK)]], rows, sem)

        def gwait(rows, sem):
            pltpu.make_async_copy(table_sh.at[idx_all.at[pl.ds(0, CK)]], rows, sem).wait()

        def owait(out, sem, outb=outb):
            pltpu.make_async_copy(out, outb.at[pl.ds(0, C)], sem).wait()

        def compute(t, rows, out):
            @pl.loop(0, C)
            def _nodes(i):
                e_vec = e_all[pl.ds((t * C + i) * K, K)]
                es = [e_vec[k] for k in range(K)]
                for f8 in range(8):
                    sl = pl.ds(f8 * 16, 16)
                    prods = [es[k] * rows[i * K + k, sl] for k in range(K)]
                    while len(prods) > 1:
                        prods = [prods[j] + prods[j + 1]
                                 for j in range(0, len(prods), 2)]
                    out[i, sl] = prods[0]

        def run(nchunk, idxf=idxf, ef=ef, outb=outb):
            pltpu.sync_copy(idxf.at[pl.ds(base * K, nchunk * CK)],
                            idx_all.at[pl.ds(0, nchunk * CK)])
            pltpu.sync_copy(ef.at[pl.ds(base * K, nchunk * CK)],
                            e_all.at[pl.ds(0, nchunk * CK)])
            gather(0, rows_a, gsem_a)

            @pl.loop(0, nchunk // 2)
            def _t2(t2):
                t = t2 * 2
                gather(t + 1, rows_b, gsem_b)
                gwait(rows_a, gsem_a)

                @pl.when(t2 > 0)
                def _():
                    owait(out_a, osem_a)

                compute(t, rows_a, out_a)
                pltpu.async_copy(out_a, outb.at[pl.ds(base + t * C, C)], osem_a)

                @pl.when(t2 + 1 < nchunk // 2)
                def _():
                    gather(t + 2, rows_a, gsem_a)

                gwait(rows_b, gsem_b)

                @pl.when(t2 > 0)
                def _():
                    owait(out_b, osem_b)

                compute(t + 1, rows_b, out_b)
                pltpu.async_copy(out_b, outb.at[pl.ds(base + (t + 1) * C, C)], osem_b)

            owait(out_a, osem_a)
            owait(out_b, osem_b)

        @pl.when(wid < NW - 1)
        def _full():
            run(NCHUNK)

        @pl.when(wid == NW - 1)
        def _tail():
            run((N - (NW - 1) * NODES_W) // C)


def _bags(vi, vn, nhi_f, ini_f, nhe_f, ine_f):
    call = functools.partial(
        pl.kernel,
        out_type=[jax.ShapeDtypeStruct((NP, V), jnp.float32)] * 2,
        mesh=plsc.VectorSubcoreMesh(core_axis_name="c", subcore_axis_name="s"),
        scratch_types=[
            pltpu.VMEM((NODES_W * K,), jnp.int32),
            pltpu.VMEM((NODES_W * K,), jnp.float32),
            pltpu.VMEM((CK, V), jnp.float32),
            pltpu.VMEM((CK, V), jnp.float32),
            pltpu.VMEM((C, V), jnp.float32),
            pltpu.VMEM((C, V), jnp.float32),
            pltpu.VMEM_SHARED((NP, V), jnp.float32),
            pltpu.SemaphoreType.DMA,
            pltpu.SemaphoreType.DMA,
            pltpu.SemaphoreType.DMA,
            pltpu.SemaphoreType.DMA,
        ],
    )(_bag_body)
    return call(vi, vn, nhi_f, ini_f, nhe_f, ine_f)


def _egcn_body(vi_ref, vn_ref, bi_ref, bn_ref, w_ref, hi_ref, hn_ref):
    w = w_ref[...]
    for v_ref, b_ref, h_ref in ((vi_ref, bi_ref, hi_ref),
                                (vn_ref, bn_ref, hn_ref)):
        z = jnp.dot(v_ref[...].astype(jnp.bfloat16), w,
                    preferred_element_type=jnp.float32)
        g = jnp.dot(b_ref[...].astype(jnp.bfloat16), w,
                    preferred_element_type=jnp.float32)
        p = z * g
        h = p[:, 0:V]
        for f in range(1, F):
            h = h + p[:, f * V:(f + 1) * V]
        h_ref[...] = jnp.where(h >= 0.0, h, 0.01 * h)


def _egcn_dense(vi, vn, bag_i, bag_n, wr):
    return pl.pallas_call(
        _egcn_body,
        grid=(NP // BM,),
        in_specs=[
            pl.BlockSpec((BM, V), lambda i: (i, 0)),
            pl.BlockSpec((BM, V), lambda i: (i, 0)),
            pl.BlockSpec((BM, V), lambda i: (i, 0)),
            pl.BlockSpec((BM, V), lambda i: (i, 0)),
            pl.BlockSpec((V, D), lambda i: (0, 0)),
        ],
        out_specs=[
            pl.BlockSpec((BM, V), lambda i: (i, 0)),
            pl.BlockSpec((BM, V), lambda i: (i, 0)),
        ],
        out_shape=[jax.ShapeDtypeStruct((N, V), jnp.float32)] * 2,
    )(vi, vn, bag_i, bag_n, wr)


def kernel(vertices_int, vertices_nh, nh_indices, int_indices,
           nh_edges, int_edges, is_int, W):
    nh_e = nh_edges[..., None] if nh_edges.ndim != 3 else nh_edges
    int_e = int_edges[..., None] if int_edges.ndim != 3 else int_edges

    nhi_f = nh_indices.astype(jnp.int32).reshape(N * K)
    ini_f = int_indices.astype(jnp.int32).reshape(N * K)
    nhe_f = nh_edges.astype(jnp.float32).reshape(N * K)
    ine_f = int_edges.astype(jnp.float32).reshape(N * K)
    wr = jnp.transpose(W, (1, 2, 0)).reshape(V, D).astype(jnp.bfloat16)

    vi, vn = _masked_vertices(vertices_int.astype(jnp.float32),
                              vertices_nh.astype(jnp.float32),
                              is_int.astype(jnp.int32))
    bag_i, bag_n = _bags(vi, vn, nhi_f, ini_f, nhe_f, ine_f)
    h_int, h_nh = _egcn_dense(vi, vn, bag_i, bag_n, wr)

    return (h_int, h_nh, nh_indices, int_indices, nh_e, int_e, is_int)


# speedup vs baseline: 1.0796x; 1.0796x over previous
import functools

import jax
import jax.numpy as jnp
from jax import lax
from jax.experimental import pallas as pl
from jax.experimental.pallas import tpu as pltpu
from jax.experimental.pallas import tpu_sc as plsc

N = 10000
K = 16
V = 128
F = 16
D = V * F
NW = 32
NP = 10240
NODES_W = NP // NW
C = 8
CK = C * K
NCHUNK = NODES_W // C
BM = 512


def _mask_body(vint_ref, vnh_ref, isint_ref, vi_ref, vn_ref):
    m = (isint_ref[...] == 1).astype(jnp.float32)
    vi_ref[...] = vint_ref[...] * m
    vn_ref[...] = vnh_ref[...] * (1.0 - m)


def _masked_vertices(vint, vnh, isint):
    return pl.pallas_call(
        _mask_body,
        grid=(NP // BM,),
        in_specs=[
            pl.BlockSpec((BM, V), lambda i: (i, 0)),
            pl.BlockSpec((BM, V), lambda i: (i, 0)),
            pl.BlockSpec((BM, 1), lambda i: (i, 0)),
        ],
        out_specs=[
            pl.BlockSpec((BM, V), lambda i: (i, 0)),
            pl.BlockSpec((BM, V), lambda i: (i, 0)),
        ],
        out_shape=[jax.ShapeDtypeStruct((NP, V), jnp.float32)] * 2,
    )(vint, vnh, isint)


def _bag_body(vi_hbm, vn_hbm, nhi_hbm, ini_hbm, nhe_hbm, ine_hbm,
              bag_i_hbm, bag_n_hbm, idx_all, e_all, rows_a, rows_b,
              out_a, out_b, table_sh, gsem_a, gsem_b, osem_a, osem_b):
    cid = lax.axis_index("c")
    sid = lax.axis_index("s")
    wid = sid * 2 + cid
    base = wid * NODES_W
    rpt = NP // 16
    for idxf, ef, tab, outb in ((nhi_hbm, nhe_hbm, vi_hbm, bag_i_hbm),
                                (ini_hbm, ine_hbm, vn_hbm, bag_n_hbm)):
        plsc.subcore_barrier()
        pltpu.sync_copy(tab.at[pl.ds(sid * rpt, rpt)],
                        table_sh.at[pl.ds(sid * rpt, rpt)])
        plsc.subcore_barrier()

        def gather(t, rows, sem):
            pltpu.async_copy(table_sh.at[idx_all.at[pl.ds(t * CK, CK)]], rows, sem)

        def gwait(rows, sem):
            pltpu.make_async_copy(table_sh.at[idx_all.at[pl.ds(0, CK)]], rows, sem).wait()

        def owait(out, sem, outb=outb):
            pltpu.make_async_copy(out, outb.at[pl.ds(0, C)], sem).wait()

        def compute(t, rows, out):
            @plsc.parallel_loop(0, C)
            def _nodes(i):
                e_vec = e_all[pl.ds((t * C + i) * K, K)]
                es = [e_vec[k] for k in range(K)]
                for f8 in range(8):
                    sl = pl.ds(f8 * 16, 16)
                    prods = [es[k] * rows[i * K + k, sl] for k in range(K)]
                    while len(prods) > 1:
                        prods = [prods[j] + prods[j + 1]
                                 for j in range(0, len(prods), 2)]
                    out[i, sl] = prods[0]

        def run(nchunk, idxf=idxf, ef=ef, outb=outb):
            pltpu.sync_copy(idxf.at[pl.ds(base * K, nchunk * CK)],
                            idx_all.at[pl.ds(0, nchunk * CK)])
            pltpu.sync_copy(ef.at[pl.ds(base * K, nchunk * CK)],
                            e_all.at[pl.ds(0, nchunk * CK)])
            gather(0, rows_a, gsem_a)

            @pl.loop(0, nchunk // 2)
            def _t2(t2):
                t = t2 * 2
                gather(t + 1, rows_b, gsem_b)
                gwait(rows_a, gsem_a)

                @pl.when(t2 > 0)
                def _():
                    owait(out_a, osem_a)

                compute(t, rows_a, out_a)
                pltpu.async_copy(out_a, outb.at[pl.ds(base + t * C, C)], osem_a)

                @pl.when(t2 + 1 < nchunk // 2)
                def _():
                    gather(t + 2, rows_a, gsem_a)

                gwait(rows_b, gsem_b)

                @pl.when(t2 > 0)
                def _():
                    owait(out_b, osem_b)

                compute(t + 1, rows_b, out_b)
                pltpu.async_copy(out_b, outb.at[pl.ds(base + (t + 1) * C, C)], osem_b)

            owait(out_a, osem_a)
            owait(out_b, osem_b)

        @pl.when(wid < NW - 1)
        def _full():
            run(NCHUNK)

        @pl.when(wid == NW - 1)
        def _tail():
            run((N - (NW - 1) * NODES_W) // C)


def _bags(vi, vn, nhi_f, ini_f, nhe_f, ine_f):
    call = functools.partial(
        pl.kernel,
        out_type=[jax.ShapeDtypeStruct((NP, V), jnp.float32)] * 2,
        mesh=plsc.VectorSubcoreMesh(core_axis_name="c", subcore_axis_name="s"),
        scratch_types=[
            pltpu.VMEM((NODES_W * K,), jnp.int32),
            pltpu.VMEM((NODES_W * K,), jnp.float32),
            pltpu.VMEM((CK, V), jnp.float32),
            pltpu.VMEM((CK, V), jnp.float32),
            pltpu.VMEM((C, V), jnp.float32),
            pltpu.VMEM((C, V), jnp.float32),
            pltpu.VMEM_SHARED((NP, V), jnp.float32),
            pltpu.SemaphoreType.DMA,
            pltpu.SemaphoreType.DMA,
            pltpu.SemaphoreType.DMA,
            pltpu.SemaphoreType.DMA,
        ],
    )(_bag_body)
    return call(vi, vn, nhi_f, ini_f, nhe_f, ine_f)


def _egcn_body(vi_ref, vn_ref, bi_ref, bn_ref, w_ref, hi_ref, hn_ref):
    w = w_ref[...]
    for v_ref, b_ref, h_ref in ((vi_ref, bi_ref, hi_ref),
                                (vn_ref, bn_ref, hn_ref)):
        z = jnp.dot(v_ref[...].astype(jnp.bfloat16), w,
                    preferred_element_type=jnp.float32)
        g = jnp.dot(b_ref[...].astype(jnp.bfloat16), w,
                    preferred_element_type=jnp.float32)
        p = z * g
        h = p[:, 0:V]
        for f in range(1, F):
            h = h + p[:, f * V:(f + 1) * V]
        h_ref[...] = jnp.where(h >= 0.0, h, 0.01 * h)


def _egcn_dense(vi, vn, bag_i, bag_n, wr):
    return pl.pallas_call(
        _egcn_body,
        grid=(NP // BM,),
        in_specs=[
            pl.BlockSpec((BM, V), lambda i: (i, 0)),
            pl.BlockSpec((BM, V), lambda i: (i, 0)),
            pl.BlockSpec((BM, V), lambda i: (i, 0)),
            pl.BlockSpec((BM, V), lambda i: (i, 0)),
            pl.BlockSpec((V, D), lambda i: (0, 0)),
        ],
        out_specs=[
            pl.BlockSpec((BM, V), lambda i: (i, 0)),
            pl.BlockSpec((BM, V), lambda i: (i, 0)),
        ],
        out_shape=[jax.ShapeDtypeStruct((N, V), jnp.float32)] * 2,
    )(vi, vn, bag_i, bag_n, wr)


def kernel(vertices_int, vertices_nh, nh_indices, int_indices,
           nh_edges, int_edges, is_int, W):
    nh_e = nh_edges[..., None] if nh_edges.ndim != 3 else nh_edges
    int_e = int_edges[..., None] if int_edges.ndim != 3 else int_edges

    nhi_f = nh_indices.astype(jnp.int32).reshape(N * K)
    ini_f = int_indices.astype(jnp.int32).reshape(N * K)
    nhe_f = nh_edges.astype(jnp.float32).reshape(N * K)
    ine_f = int_edges.astype(jnp.float32).reshape(N * K)
    wr = jnp.transpose(W, (1, 2, 0)).reshape(V, D).astype(jnp.bfloat16)

    vi, vn = _masked_vertices(vertices_int.astype(jnp.float32),
                              vertices_nh.astype(jnp.float32),
                              is_int.astype(jnp.int32))
    bag_i, bag_n = _bags(vi, vn, nhi_f, ini_f, nhe_f, ine_f)
    h_int, h_nh = _egcn_dense(vi, vn, bag_i, bag_n, wr)

    return (h_int, h_nh, nh_indices, int_indices, nh_e, int_e, is_int)
